# initial kernel scaffold (unmeasured)
import jax
import jax.numpy as jnp
from jax import lax
from jax.experimental import pallas as pl
from jax.experimental.pallas import tpu as pltpu

B = 32
H = 16
D = 128
PAGE = 32
P_LOCAL = 256
NB = 32
BP = P_LOCAL // NB
TPB = BP * PAGE
NDEV_Y = 4


def _body(q_ref, k_ref, v_ref, wt_ref, out_ref,
          kbuf, vbuf, acc_comm, l_comm,
          sem_k, sem_v, asend, arecv, lsend, lrecv):
    my_x = lax.axis_index("x")
    my_y = lax.axis_index("y")
    my_z = lax.axis_index("z")

    acc_comm[0, :, :, :] = jnp.zeros((B, H, D), jnp.float32)
    l_comm[0, :, :] = jnp.zeros((B, H), jnp.float32)

    def k_dma(b, slot):
        return pltpu.make_async_copy(
            k_ref.at[pl.ds(b * BP, BP)], kbuf.at[slot], sem_k.at[slot])

    def v_dma(b, slot):
        return pltpu.make_async_copy(
            v_ref.at[pl.ds(b * BP, BP)], vbuf.at[slot], sem_v.at[slot])

    k_dma(0, 0).start()
    v_dma(0, 0).start()

    def loop_body(b, carry):
        slot = lax.rem(b, 2)
        nslot = lax.rem(b + 1, 2)

        @pl.when(b + 1 < NB)
        def _():
            k_dma(b + 1, nslot).start()
            v_dma(b + 1, nslot).start()

        k_dma(b, slot).wait()
        v_dma(b, slot).wait()

        wt_b = wt_ref[b]
        kblk = kbuf.at[slot]
        vblk = vbuf.at[slot]
        for h in range(H):
            kh = kblk[:, :, h, :].reshape(TPB, D).astype(jnp.bfloat16)
            vh = vblk[:, :, h, :].reshape(TPB, D).astype(jnp.bfloat16)
            qh = q_ref[:, h, :]
            s = lax.dot_general(
                qh, kh, (((1,), (1,)), ((), ())),
                preferred_element_type=jnp.float32)
            e = jnp.exp(s) * wt_b
            l_comm[0, :, h:h + 1] = (
                l_comm[0, :, h:h + 1] + jnp.sum(e, axis=1, keepdims=True))
            pv = lax.dot_general(
                e.astype(jnp.bfloat16), vh, (((1,), (0,)), ((), ())),
                preferred_element_type=jnp.float32)
            acc_comm[0, :, h, :] = acc_comm[0, :, h, :] + pv
        return carry

    lax.fori_loop(0, NB, loop_body, 0)

    left = lax.rem(my_y - 1 + NDEV_Y, NDEV_Y)
    right = lax.rem(my_y + 1, NDEV_Y)
    barrier = pltpu.get_barrier_semaphore()
    for nbr in (left, right):
        pl.semaphore_signal(
            barrier, inc=1,
            device_id=(my_x, nbr, my_z),
            device_id_type=pl.DeviceIdType.MESH)
    pl.semaphore_wait(barrier, 2)

    for hop in range(NDEV_Y - 1):
        a = pltpu.make_async_remote_copy(
            src_ref=acc_comm.at[hop], dst_ref=acc_comm.at[hop + 1],
            send_sem=asend.at[hop], recv_sem=arecv.at[hop],
            device_id=(my_x, right, my_z),
            device_id_type=pl.DeviceIdType.MESH)
        lr = pltpu.make_async_remote_copy(
            src_ref=l_comm.at[hop], dst_ref=l_comm.at[hop + 1],
            send_sem=lsend.at[hop], recv_sem=lrecv.at[hop],
            device_id=(my_x, right, my_z),
            device_id_type=pl.DeviceIdType.MESH)
        a.start()
        lr.start()
        a.wait()
        lr.wait()

    acc_tot = (acc_comm[0, :, :, :] + acc_comm[1, :, :, :]
               + acc_comm[2, :, :, :] + acc_comm[3, :, :, :])
    l_tot = (l_comm[0, :, :] + l_comm[1, :, :]
             + l_comm[2, :, :] + l_comm[3, :, :])
    out_ref[:, 0, :, :] = acc_tot / l_tot[:, :, None]


def kernel(Q, K, V, bt, lens):
    my_y = lax.axis_index("y")
    base = my_y * P_LOCAL
    pid = base + jnp.arange(P_LOCAL, dtype=jnp.int32)
    valid = jnp.arange(bt.shape[1], dtype=jnp.int32)[None, :] < lens[:, None]
    w = jnp.sum(
        (bt[:, :, None] == pid[None, None, :]) & valid[:, :, None],
        axis=1).astype(jnp.float32)
    wt = jnp.repeat(w, PAGE, axis=1)
    wt = wt.reshape(B, NB, TPB).swapaxes(0, 1)
    qb = (Q.reshape(B, H, D) * (D ** -0.5)).astype(jnp.bfloat16)

    return pl.pallas_call(
        _body,
        out_shape=jax.ShapeDtypeStruct((B, 1, H, D), jnp.float32),
        in_specs=[
            pl.BlockSpec(memory_space=pltpu.VMEM),
            pl.BlockSpec(memory_space=pltpu.ANY),
            pl.BlockSpec(memory_space=pltpu.ANY),
            pl.BlockSpec(memory_space=pltpu.VMEM),
        ],
        out_specs=pl.BlockSpec(memory_space=pltpu.VMEM),
        scratch_shapes=[
            pltpu.VMEM((2, BP, PAGE, H, D), jnp.float32),
            pltpu.VMEM((2, BP, PAGE, H, D), jnp.float32),
            pltpu.VMEM((NDEV_Y, B, H, D), jnp.float32),
            pltpu.VMEM((NDEV_Y, B, H), jnp.float32),
            pltpu.SemaphoreType.DMA((2,)),
            pltpu.SemaphoreType.DMA((2,)),
            pltpu.SemaphoreType.DMA((NDEV_Y - 1,)),
            pltpu.SemaphoreType.DMA((NDEV_Y - 1,)),
            pltpu.SemaphoreType.DMA((NDEV_Y - 1,)),
            pltpu.SemaphoreType.DMA((NDEV_Y - 1,)),
        ],
        compiler_params=pltpu.CompilerParams(collective_id=0),
    )(qb, K, V, wt)


# baseline (device time: 181335 ns/iter reference)
import jax
import jax.numpy as jnp
from jax import lax
from jax.experimental import pallas as pl
from jax.experimental.pallas import tpu as pltpu

B = 32
H = 16
D = 128
PAGE = 32
P_LOCAL = 256
NB = 32
BP = P_LOCAL // NB
TPB = BP * PAGE
NDEV_Y = 4


def _body(q_ref, k_ref, v_ref, wt_ref, out_ref,
          kbuf, vbuf, acc_comm, l_comm,
          sem_k, sem_v, asend, arecv, lsend, lrecv):
    my_x = lax.axis_index("x")
    my_y = lax.axis_index("y")
    my_z = lax.axis_index("z")

    acc_comm[0, :, :, :] = jnp.zeros((B, H, D), jnp.float32)
    l_comm[0, :, :] = jnp.zeros((B, H), jnp.float32)

    def k_dma(b, slot):
        return pltpu.make_async_copy(
            k_ref.at[pl.ds(b * BP, BP)], kbuf.at[slot], sem_k.at[slot])

    def v_dma(b, slot):
        return pltpu.make_async_copy(
            v_ref.at[pl.ds(b * BP, BP)], vbuf.at[slot], sem_v.at[slot])

    k_dma(0, 0).start()
    v_dma(0, 0).start()

    def loop_body(b, carry):
        slot = lax.rem(b, 2)
        nslot = lax.rem(b + 1, 2)

        @pl.when(b + 1 < NB)
        def _():
            k_dma(b + 1, nslot).start()
            v_dma(b + 1, nslot).start()

        k_dma(b, slot).wait()
        v_dma(b, slot).wait()

        wt_b = wt_ref[b]
        kblk = kbuf.at[slot]
        vblk = vbuf.at[slot]
        for h in range(H):
            kh = kblk[:, :, h, :].reshape(TPB, D).astype(jnp.bfloat16)
            vh = vblk[:, :, h, :].reshape(TPB, D).astype(jnp.bfloat16)
            qh = q_ref[:, h, :]
            s = lax.dot_general(
                qh, kh, (((1,), (1,)), ((), ())),
                preferred_element_type=jnp.float32)
            e = jnp.exp(s) * wt_b
            l_comm[0, :, h:h + 1] = (
                l_comm[0, :, h:h + 1] + jnp.sum(e, axis=1, keepdims=True))
            pv = lax.dot_general(
                e.astype(jnp.bfloat16), vh, (((1,), (0,)), ((), ())),
                preferred_element_type=jnp.float32)
            acc_comm[0, :, h, :] = acc_comm[0, :, h, :] + pv
        return carry

    lax.fori_loop(0, NB, loop_body, 0)

    left = lax.rem(my_y - 1 + NDEV_Y, NDEV_Y)
    right = lax.rem(my_y + 1, NDEV_Y)
    barrier = pltpu.get_barrier_semaphore()
    for nbr in (left, right):
        pl.semaphore_signal(
            barrier, inc=1,
            device_id=(my_x, nbr, my_z),
            device_id_type=pl.DeviceIdType.MESH)
    pl.semaphore_wait(barrier, 2)

    for hop in range(NDEV_Y - 1):
        a = pltpu.make_async_remote_copy(
            src_ref=acc_comm.at[hop], dst_ref=acc_comm.at[hop + 1],
            send_sem=asend.at[hop], recv_sem=arecv.at[hop],
            device_id=(my_x, right, my_z),
            device_id_type=pl.DeviceIdType.MESH)
        lr = pltpu.make_async_remote_copy(
            src_ref=l_comm.at[hop], dst_ref=l_comm.at[hop + 1],
            send_sem=lsend.at[hop], recv_sem=lrecv.at[hop],
            device_id=(my_x, right, my_z),
            device_id_type=pl.DeviceIdType.MESH)
        a.start()
        lr.start()
        a.wait()
        lr.wait()

    acc_tot = (acc_comm[0, :, :, :] + acc_comm[1, :, :, :]
               + acc_comm[2, :, :, :] + acc_comm[3, :, :, :])
    l_tot = (l_comm[0, :, :] + l_comm[1, :, :]
             + l_comm[2, :, :] + l_comm[3, :, :])
    out_ref[:, 0, :, :] = acc_tot / l_tot[:, :, None]


def kernel(Q, K, V, bt, lens):
    my_y = lax.axis_index("y")
    base = my_y * P_LOCAL
    pid = base + jnp.arange(P_LOCAL, dtype=jnp.int32)
    valid = jnp.arange(bt.shape[1], dtype=jnp.int32)[None, :] < lens[:, None]
    w = jnp.sum(
        (bt[:, :, None] == pid[None, None, :]) & valid[:, :, None],
        axis=1).astype(jnp.float32)
    wt = jnp.repeat(w, PAGE, axis=1)
    wt = wt.reshape(B, NB, TPB).swapaxes(0, 1)
    qb = (Q.reshape(B, H, D) * (D ** -0.5)).astype(jnp.bfloat16)

    return pl.pallas_call(
        _body,
        out_shape=jax.ShapeDtypeStruct((B, 1, H, D), jnp.float32),
        in_specs=[
            pl.BlockSpec(memory_space=pltpu.VMEM),
            pl.BlockSpec(memory_space=pl.ANY),
            pl.BlockSpec(memory_space=pl.ANY),
            pl.BlockSpec(memory_space=pltpu.VMEM),
        ],
        out_specs=pl.BlockSpec(memory_space=pltpu.VMEM),
        scratch_shapes=[
            pltpu.VMEM((2, BP, PAGE, H, D), jnp.float32),
            pltpu.VMEM((2, BP, PAGE, H, D), jnp.float32),
            pltpu.VMEM((NDEV_Y, B, H, D), jnp.float32),
            pltpu.VMEM((NDEV_Y, B, H), jnp.float32),
            pltpu.SemaphoreType.DMA((2,)),
            pltpu.SemaphoreType.DMA((2,)),
            pltpu.SemaphoreType.DMA((NDEV_Y - 1,)),
            pltpu.SemaphoreType.DMA((NDEV_Y - 1,)),
            pltpu.SemaphoreType.DMA((NDEV_Y - 1,)),
            pltpu.SemaphoreType.DMA((NDEV_Y - 1,)),
        ],
        compiler_params=pltpu.CompilerParams(collective_id=0),
    )(qb, K, V, wt)


# device time: 176413 ns/iter; 1.0279x vs baseline; 1.0279x over previous
import jax
import jax.numpy as jnp
from jax import lax
from jax.experimental import pallas as pl
from jax.experimental.pallas import tpu as pltpu

B = 32
H = 16
D = 128
PAGE = 32
P_LOCAL = 256
NB = 16
BP = P_LOCAL // NB
TPB = BP * PAGE
NDEV_Y = 4


def _body(q_ref, k_ref, v_ref, wt_ref, out_ref,
          kbuf, vbuf, acc_comm, l_comm,
          sem_k, sem_v, asend, arecv, lsend, lrecv):
    my_x = lax.axis_index("x")
    my_y = lax.axis_index("y")
    my_z = lax.axis_index("z")

    acc_comm[0, :, :, :] = jnp.zeros((B, H, D), jnp.float32)
    l_comm[0, :, :] = jnp.zeros((B, H), jnp.float32)

    def k_dma(b, slot):
        return pltpu.make_async_copy(
            k_ref.at[pl.ds(b * BP, BP)], kbuf.at[slot], sem_k.at[slot])

    def v_dma(b, slot):
        return pltpu.make_async_copy(
            v_ref.at[pl.ds(b * BP, BP)], vbuf.at[slot], sem_v.at[slot])

    k_dma(0, 0).start()
    v_dma(0, 0).start()

    def loop_body(b, carry):
        slot = lax.rem(b, 2)
        nslot = lax.rem(b + 1, 2)

        @pl.when(b + 1 < NB)
        def _():
            k_dma(b + 1, nslot).start()
            v_dma(b + 1, nslot).start()

        k_dma(b, slot).wait()
        v_dma(b, slot).wait()

        wt_b = wt_ref[b]
        kblk = kbuf.at[slot]
        vblk = vbuf.at[slot]
        for h in range(H):
            kh = kblk[:, :, h, :].reshape(TPB, D).astype(jnp.bfloat16)
            vh = vblk[:, :, h, :].reshape(TPB, D).astype(jnp.bfloat16)
            qh = q_ref[:, h, :]
            s = lax.dot_general(
                qh, kh, (((1,), (1,)), ((), ())),
                preferred_element_type=jnp.float32)
            e = jnp.exp(s) * wt_b
            l_comm[0, :, h:h + 1] = (
                l_comm[0, :, h:h + 1] + jnp.sum(e, axis=1, keepdims=True))
            pv = lax.dot_general(
                e.astype(jnp.bfloat16), vh, (((1,), (0,)), ((), ())),
                preferred_element_type=jnp.float32)
            acc_comm[0, :, h, :] = acc_comm[0, :, h, :] + pv
        return carry

    lax.fori_loop(0, NB, loop_body, 0)

    left = lax.rem(my_y - 1 + NDEV_Y, NDEV_Y)
    right = lax.rem(my_y + 1, NDEV_Y)
    barrier = pltpu.get_barrier_semaphore()
    for nbr in (left, right):
        pl.semaphore_signal(
            barrier, inc=1,
            device_id=(my_x, nbr, my_z),
            device_id_type=pl.DeviceIdType.MESH)
    pl.semaphore_wait(barrier, 2)

    for hop in range(NDEV_Y - 1):
        a = pltpu.make_async_remote_copy(
            src_ref=acc_comm.at[hop], dst_ref=acc_comm.at[hop + 1],
            send_sem=asend.at[hop], recv_sem=arecv.at[hop],
            device_id=(my_x, right, my_z),
            device_id_type=pl.DeviceIdType.MESH)
        lr = pltpu.make_async_remote_copy(
            src_ref=l_comm.at[hop], dst_ref=l_comm.at[hop + 1],
            send_sem=lsend.at[hop], recv_sem=lrecv.at[hop],
            device_id=(my_x, right, my_z),
            device_id_type=pl.DeviceIdType.MESH)
        a.start()
        lr.start()
        a.wait()
        lr.wait()

    acc_tot = (acc_comm[0, :, :, :] + acc_comm[1, :, :, :]
               + acc_comm[2, :, :, :] + acc_comm[3, :, :, :])
    l_tot = (l_comm[0, :, :] + l_comm[1, :, :]
             + l_comm[2, :, :] + l_comm[3, :, :])
    out_ref[:, 0, :, :] = acc_tot / l_tot[:, :, None]


def kernel(Q, K, V, bt, lens):
    my_y = lax.axis_index("y")
    base = my_y * P_LOCAL
    pid = base + jnp.arange(P_LOCAL, dtype=jnp.int32)
    valid = jnp.arange(bt.shape[1], dtype=jnp.int32)[None, :] < lens[:, None]
    w = jnp.sum(
        (bt[:, :, None] == pid[None, None, :]) & valid[:, :, None],
        axis=1).astype(jnp.float32)
    wt = jnp.repeat(w, PAGE, axis=1)
    wt = wt.reshape(B, NB, TPB).swapaxes(0, 1)
    qb = (Q.reshape(B, H, D) * (D ** -0.5)).astype(jnp.bfloat16)

    return pl.pallas_call(
        _body,
        out_shape=jax.ShapeDtypeStruct((B, 1, H, D), jnp.float32),
        in_specs=[
            pl.BlockSpec(memory_space=pltpu.VMEM),
            pl.BlockSpec(memory_space=pl.ANY),
            pl.BlockSpec(memory_space=pl.ANY),
            pl.BlockSpec(memory_space=pltpu.VMEM),
        ],
        out_specs=pl.BlockSpec(memory_space=pltpu.VMEM),
        scratch_shapes=[
            pltpu.VMEM((2, BP, PAGE, H, D), jnp.float32),
            pltpu.VMEM((2, BP, PAGE, H, D), jnp.float32),
            pltpu.VMEM((NDEV_Y, B, H, D), jnp.float32),
            pltpu.VMEM((NDEV_Y, B, H), jnp.float32),
            pltpu.SemaphoreType.DMA((2,)),
            pltpu.SemaphoreType.DMA((2,)),
            pltpu.SemaphoreType.DMA((NDEV_Y - 1,)),
            pltpu.SemaphoreType.DMA((NDEV_Y - 1,)),
            pltpu.SemaphoreType.DMA((NDEV_Y - 1,)),
            pltpu.SemaphoreType.DMA((NDEV_Y - 1,)),
        ],
        compiler_params=pltpu.CompilerParams(collective_id=0),
    )(qb, K, V, wt)


# device time: 82306 ns/iter; 2.2032x vs baseline; 2.1434x over previous
import jax
import jax.numpy as jnp
from jax import lax
from jax.experimental import pallas as pl
from jax.experimental.pallas import tpu as pltpu

B = 32
H = 16
D = 128
PAGE = 32
P_LOCAL = 256
NB = 16
BP = P_LOCAL // NB
TPB = BP * PAGE
NDEV_Y = 4


def _body(q_ref, k_ref, v_ref, wt_ref, out_ref,
          kbuf, vbuf, acc_comm, l_comm,
          sem_k, sem_v, asend, arecv, lsend, lrecv):
    my_x = lax.axis_index("x")
    my_y = lax.axis_index("y")
    my_z = lax.axis_index("z")

    acc_comm[0, :, :, :] = jnp.zeros((B, H, D), jnp.float32)
    l_comm[0, :, :] = jnp.zeros((B, H), jnp.float32)

    def k_dma(b, slot, h):
        return pltpu.make_async_copy(
            k_ref.at[pl.ds(b * BP, BP), :, h, :], kbuf.at[slot, h],
            sem_k.at[slot])

    def v_dma(b, slot, h):
        return pltpu.make_async_copy(
            v_ref.at[pl.ds(b * BP, BP), :, h, :], vbuf.at[slot, h],
            sem_v.at[slot])

    def start_block(b, slot):
        for h in range(H):
            k_dma(b, slot, h).start()
            v_dma(b, slot, h).start()

    def wait_block(b, slot):
        for h in range(H):
            k_dma(b, slot, h).wait()
            v_dma(b, slot, h).wait()

    start_block(0, 0)

    def loop_body(b, carry):
        slot = lax.rem(b, 2)
        nslot = lax.rem(b + 1, 2)

        @pl.when(b + 1 < NB)
        def _():
            start_block(b + 1, nslot)

        wait_block(b, slot)

        wt_b = wt_ref[b]
        for h in range(H):
            kh = kbuf.at[slot, h][...].reshape(TPB, D).astype(jnp.bfloat16)
            vh = vbuf.at[slot, h][...].reshape(TPB, D).astype(jnp.bfloat16)
            qh = q_ref[:, h, :]
            s = lax.dot_general(
                qh, kh, (((1,), (1,)), ((), ())),
                preferred_element_type=jnp.float32)
            e = jnp.exp(s) * wt_b
            l_comm[0, :, h:h + 1] = (
                l_comm[0, :, h:h + 1] + jnp.sum(e, axis=1, keepdims=True))
            pv = lax.dot_general(
                e.astype(jnp.bfloat16), vh, (((1,), (0,)), ((), ())),
                preferred_element_type=jnp.float32)
            acc_comm[0, :, h, :] = acc_comm[0, :, h, :] + pv
        return carry

    lax.fori_loop(0, NB, loop_body, 0)

    left = lax.rem(my_y - 1 + NDEV_Y, NDEV_Y)
    right = lax.rem(my_y + 1, NDEV_Y)
    barrier = pltpu.get_barrier_semaphore()
    for nbr in (left, right):
        pl.semaphore_signal(
            barrier, inc=1,
            device_id=(my_x, nbr, my_z),
            device_id_type=pl.DeviceIdType.MESH)
    pl.semaphore_wait(barrier, 2)

    for hop in range(NDEV_Y - 1):
        a = pltpu.make_async_remote_copy(
            src_ref=acc_comm.at[hop], dst_ref=acc_comm.at[hop + 1],
            send_sem=asend.at[hop], recv_sem=arecv.at[hop],
            device_id=(my_x, right, my_z),
            device_id_type=pl.DeviceIdType.MESH)
        lr = pltpu.make_async_remote_copy(
            src_ref=l_comm.at[hop], dst_ref=l_comm.at[hop + 1],
            send_sem=lsend.at[hop], recv_sem=lrecv.at[hop],
            device_id=(my_x, right, my_z),
            device_id_type=pl.DeviceIdType.MESH)
        a.start()
        lr.start()
        a.wait()
        lr.wait()

    acc_tot = (acc_comm[0, :, :, :] + acc_comm[1, :, :, :]
               + acc_comm[2, :, :, :] + acc_comm[3, :, :, :])
    l_tot = (l_comm[0, :, :] + l_comm[1, :, :]
             + l_comm[2, :, :] + l_comm[3, :, :])
    out_ref[:, 0, :, :] = acc_tot / l_tot[:, :, None]


def kernel(Q, K, V, bt, lens):
    my_y = lax.axis_index("y")
    base = my_y * P_LOCAL
    pid = base + jnp.arange(P_LOCAL, dtype=jnp.int32)
    valid = jnp.arange(bt.shape[1], dtype=jnp.int32)[None, :] < lens[:, None]
    w = jnp.sum(
        (bt[:, :, None] == pid[None, None, :]) & valid[:, :, None],
        axis=1).astype(jnp.float32)
    wt = jnp.repeat(w, PAGE, axis=1)
    wt = wt.reshape(B, NB, TPB).swapaxes(0, 1)
    qb = (Q.reshape(B, H, D) * (D ** -0.5)).astype(jnp.bfloat16)

    return pl.pallas_call(
        _body,
        out_shape=jax.ShapeDtypeStruct((B, 1, H, D), jnp.float32),
        in_specs=[
            pl.BlockSpec(memory_space=pltpu.VMEM),
            pl.BlockSpec(memory_space=pl.ANY),
            pl.BlockSpec(memory_space=pl.ANY),
            pl.BlockSpec(memory_space=pltpu.VMEM),
        ],
        out_specs=pl.BlockSpec(memory_space=pltpu.VMEM),
        scratch_shapes=[
            pltpu.VMEM((2, H, BP, PAGE, D), jnp.float32),
            pltpu.VMEM((2, H, BP, PAGE, D), jnp.float32),
            pltpu.VMEM((NDEV_Y, B, H, D), jnp.float32),
            pltpu.VMEM((NDEV_Y, B, H), jnp.float32),
            pltpu.SemaphoreType.DMA((2,)),
            pltpu.SemaphoreType.DMA((2,)),
            pltpu.SemaphoreType.DMA((NDEV_Y - 1,)),
            pltpu.SemaphoreType.DMA((NDEV_Y - 1,)),
            pltpu.SemaphoreType.DMA((NDEV_Y - 1,)),
            pltpu.SemaphoreType.DMA((NDEV_Y - 1,)),
        ],
        compiler_params=pltpu.CompilerParams(collective_id=0),
    )(qb, K, V, wt)
